# Initial kernel scaffold; baseline (speedup 1.0000x reference)
import jax
import jax.numpy as jnp
from jax.experimental import pallas as pl


def kernel(user_emb, item_emb, weight0, weight1, social_edges, social_vals, inter_edges, inter_vals):
    raise NotImplementedError("write your pallas kernel here")


# speedup vs baseline: 2.6674x; 2.6674x over previous
import functools

import jax
import jax.numpy as jnp
from jax import lax
from jax.experimental import pallas as pl
from jax.experimental.pallas import tpu as pltpu
from jax.experimental.pallas import tpu_sc as plsc

_LANES = 16


def _bcast_lane(vec, lane):
  idx = jnp.full((_LANES,), lane, jnp.int32)
  dnums = lax.GatherDimensionNumbers(
      offset_dims=(), collapsed_slice_dims=(0,), start_index_map=(0,))
  return lax.gather(vec, idx[:, None], dimension_numbers=dnums,
                    slice_sizes=(1,),
                    mode=lax.GatherScatterMode.PROMISE_IN_BOUNDS)
_CHUNK = 128
_N_SUBCORES = 16
_N_CORES = 2


def _make_spmm(n_rows, n_chunks, do_relu):
  drain = 128
  rows_per_tile = -(-n_rows // (_N_SUBCORES * drain)) * drain
  n_pad = rows_per_tile * _N_SUBCORES
  n_drain = rows_per_tile // drain

  mesh = plsc.VectorSubcoreMesh(core_axis_name="c", subcore_axis_name="s")

  @functools.partial(
      pl.kernel,
      mesh=mesh,
      out_type=(
          jax.ShapeDtypeStruct((n_pad, 128), jnp.float32),
          jax.ShapeDtypeStruct((n_pad, 128), jnp.float32),
      ),
      scratch_types=(
          pltpu.VMEM((n_chunks, _CHUNK), jnp.int32),
          pltpu.VMEM((n_chunks, _CHUNK), jnp.int32),
          pltpu.VMEM((n_chunks * _CHUNK,), jnp.float32),
          pltpu.VMEM((_CHUNK, 128), jnp.float32),
          pltpu.VMEM_SHARED((n_pad, 128), jnp.float32),
          pltpu.SemaphoreType.DMA,
      ),
  )
  def spmm(x_lo, x_hi, rows_h, cols_h, vals_h, out_lo, out_hi,
           rows_v, cols_v, vals_v, gbuf, acc, sem):
    cid = lax.axis_index("c")
    sid = lax.axis_index("s")

    pltpu.sync_copy(rows_h.at[sid], rows_v)
    pltpu.sync_copy(cols_h.at[sid], cols_v)
    pltpu.sync_copy(vals_h.at[sid], vals_v)

    zeros = jnp.zeros((_LANES,), jnp.float32)

    def zero_row(i, _):
      for q in range(128 // _LANES):
        gbuf[i, pl.ds(q * _LANES, _LANES)] = zeros
      return 0

    lax.fori_loop(0, _CHUNK, zero_row, 0)
    base_row = sid * rows_per_tile
    for t in range(n_drain):
      pltpu.sync_copy(gbuf.at[pl.ds(0, drain)],
                      acc.at[pl.ds(base_row + t * drain, drain)])
    plsc.subcore_barrier()

    def do_chunk(j, _):
      idx = cols_v.at[j]

      @pl.when(cid == 0)
      def _():
        pltpu.async_copy(x_lo.at[idx], gbuf, sem).wait()

      @pl.when(cid == 1)
      def _():
        pltpu.async_copy(x_hi.at[idx], gbuf, sem).wait()

      def scale_group(g, _):
        vals16 = vals_v[pl.ds(j * _CHUNK + g * _LANES, _LANES)]
        for e16 in range(_LANES):
          v = _bcast_lane(vals16, e16)
          row = g * _LANES + e16
          for q in range(128 // _LANES):
            sl = pl.ds(q * _LANES, _LANES)
            gbuf[row, sl] = gbuf[row, sl] * v
        return 0

      lax.fori_loop(0, _CHUNK // _LANES, scale_group, 0)
      pltpu.sync_copy(gbuf, acc.at[rows_v.at[j]], add=True)
      return 0

    lax.fori_loop(0, n_chunks, do_chunk, 0)
    plsc.subcore_barrier()

    for t in range(n_drain):
      row0 = base_row + t * drain
      pltpu.sync_copy(acc.at[pl.ds(row0, drain)], gbuf.at[pl.ds(0, drain)])
      if do_relu:
        def relu_row(i, _):
          for q in range(128 // _LANES):
            sl = pl.ds(q * _LANES, _LANES)
            gbuf[i, sl] = jnp.maximum(gbuf[i, sl], 0.0)
          return 0

        lax.fori_loop(0, drain, relu_row, 0)

      @pl.when(cid == 0)
      def _():
        pltpu.sync_copy(gbuf.at[pl.ds(0, drain)],
                        out_lo.at[pl.ds(row0, drain)])

      @pl.when(cid == 1)
      def _():
        pltpu.sync_copy(gbuf.at[pl.ds(0, drain)],
                        out_hi.at[pl.ds(row0, drain)])

  return spmm


def _dense_block(alo, ahi, blo, bhi, ulo, uhi, w_ref, olo, ohi):
  parts = (alo, ahi, blo, bhi, ulo, uhi)
  acc = jnp.zeros((alo.shape[0], 256), jnp.float32)
  for i, p in enumerate(parts):
    acc += jnp.dot(p[...], w_ref[pl.ds(i * 128, 128), :],
                   preferred_element_type=jnp.float32)
  acc = jnp.maximum(acc, 0.0)
  olo[...] = acc[:, :128]
  ohi[...] = acc[:, 128:]


def _dense(alo, ahi, blo, bhi, ulo, uhi, w):
  n = alo.shape[0]
  blk = 400
  grid = n // blk
  half = pl.BlockSpec((blk, 128), lambda i: (i, 0))
  return pl.pallas_call(
      _dense_block,
      grid=(grid,),
      in_specs=[half] * 6 + [pl.BlockSpec((768, 256), lambda i: (0, 0))],
      out_specs=[half, half],
      out_shape=(
          jax.ShapeDtypeStruct((n, 128), jnp.float32),
          jax.ShapeDtypeStruct((n, 128), jnp.float32),
      ),
  )(alo, ahi, blo, bhi, ulo, uhi, w)


def _prep_edges(rows, cols, vals, n_chunks):
  e = rows.shape[0]
  tot = _N_SUBCORES * n_chunks * _CHUNK
  pad = tot - e
  r = jnp.pad(rows.astype(jnp.int32), (0, pad)).reshape(
      _N_SUBCORES, n_chunks, _CHUNK)
  c = jnp.pad(cols.astype(jnp.int32), (0, pad)).reshape(
      _N_SUBCORES, n_chunks, _CHUNK)
  v = jnp.pad(vals, (0, pad)).reshape(_N_SUBCORES, n_chunks * _CHUNK)
  return r, c, v


def kernel(user_emb, item_emb, weight0, weight1, social_edges, social_vals,
           inter_edges, inter_vals):
  n_u, d = user_emb.shape
  n_i = item_emb.shape[0]
  e = social_edges.shape[1]
  per_tile = -(-e // _N_SUBCORES)
  n_chunks = -(-per_tile // _CHUNK)

  s_r, s_c, s_v = _prep_edges(social_edges[0], social_edges[1], social_vals,
                              n_chunks)
  i_r, i_c, i_v = _prep_edges(inter_edges[0], inter_edges[1], inter_vals,
                              n_chunks)
  t_r, t_c, t_v = _prep_edges(inter_edges[1], inter_edges[0], inter_vals,
                              n_chunks)

  spmm = _make_spmm(n_u, n_chunks, do_relu=False)
  spmm_relu = _make_spmm(n_u, n_chunks, do_relu=True)

  u_lo, u_hi = user_emb[:, :128], user_emb[:, 128:]
  it_lo, it_hi = item_emb[:, :128], item_emb[:, 128:]

  outs_u = [(u_lo, u_hi)]
  outs_i = [(it_lo, it_hi)]
  for w in (weight0, weight1):
    a_lo, a_hi = spmm(u_lo, u_hi, s_r, s_c, s_v)
    b_lo, b_hi = spmm(it_lo, it_hi, i_r, i_c, i_v)
    it2_lo, it2_hi = spmm_relu(u_lo, u_hi, t_r, t_c, t_v)
    a_lo, a_hi = a_lo[:n_u], a_hi[:n_u]
    b_lo, b_hi = b_lo[:n_u], b_hi[:n_u]
    new_it = (it2_lo[:n_i], it2_hi[:n_i])
    new_u = _dense(a_lo, a_hi, b_lo, b_hi, u_lo, u_hi, w)
    u_lo, u_hi = new_u
    it_lo, it_hi = new_it
    outs_u.append(new_u)
    outs_i.append(new_it)

  final_user = jnp.concatenate([h for pair in outs_u for h in pair], axis=1)
  final_item = jnp.concatenate([h for pair in outs_i for h in pair], axis=1)
  return final_user, final_item


# speedup vs baseline: 4.0197x; 1.5070x over previous
import functools

import jax
import jax.numpy as jnp
from jax import lax
from jax.experimental import pallas as pl
from jax.experimental.pallas import tpu as pltpu
from jax.experimental.pallas import tpu_sc as plsc

_LANES = 16
_CHUNK = 80
_N_SUBCORES = 16
_N_CORES = 2
_NBUF = 3


def _bcast_lane(vec, lane):
  idx = jnp.full((_LANES,), lane, jnp.int32)
  dnums = lax.GatherDimensionNumbers(
      offset_dims=(), collapsed_slice_dims=(0,), start_index_map=(0,))
  return lax.gather(vec, idx[:, None], dimension_numbers=dnums,
                    slice_sizes=(1,),
                    mode=lax.GatherScatterMode.PROMISE_IN_BOUNDS)


def _make_spmm(n_rows, n_chunks, do_relu):
  drain = 80
  rows_per_tile = 640
  n_pad = rows_per_tile * _N_SUBCORES
  assert n_rows <= n_pad and rows_per_tile % drain == 0
  assert n_chunks % _NBUF == 0
  n_drain = rows_per_tile // drain

  mesh = plsc.VectorSubcoreMesh(core_axis_name="c", subcore_axis_name="s")

  @functools.partial(
      pl.kernel,
      mesh=mesh,
      out_type=(
          jax.ShapeDtypeStruct((n_pad, 128), jnp.float32),
          jax.ShapeDtypeStruct((n_pad, 128), jnp.float32),
      ),
      scratch_types=(
          pltpu.VMEM((n_chunks, _CHUNK), jnp.int32),
          pltpu.VMEM((_NBUF, 2, _CHUNK), jnp.int32),
          pltpu.VMEM((_NBUF, _CHUNK, 128), jnp.float32),
          pltpu.VMEM_SHARED((n_pad, 128), jnp.float32),
          (pltpu.SemaphoreType.DMA,) * _NBUF,
          (pltpu.SemaphoreType.DMA,) * _NBUF,
          (pltpu.SemaphoreType.DMA,) * _NBUF,
      ),
  )
  def spmm(x_lo, x_hi, zeros_h, rows_h, cv_h, out_lo, out_hi,
           rows_v, cv_v, gb, acc, gsems, ssems, csems):
    cid = lax.axis_index("c")
    sid = lax.axis_index("s")

    base_row = sid * rows_per_tile
    pltpu.async_copy(zeros_h, acc.at[pl.ds(base_row, rows_per_tile)],
                     gsems[0])
    pltpu.sync_copy(rows_h.at[sid], rows_v)
    pltpu.make_async_copy(zeros_h, acc.at[pl.ds(0, rows_per_tile)],
                          gsems[0]).wait()
    plsc.subcore_barrier()

    def issue_cv(jj, b):
      pltpu.async_copy(cv_h.at[sid, jj], cv_v.at[b], csems[b])

    def wait_cv(b):
      pltpu.make_async_copy(cv_h.at[0, 0], cv_v.at[b], csems[b]).wait()

    def issue_gather(b):
      idx = cv_v.at[b, 0]

      @pl.when(cid == 0)
      def _():
        pltpu.async_copy(x_lo.at[idx], gb.at[b], gsems[b])

      @pl.when(cid == 1)
      def _():
        pltpu.async_copy(x_hi.at[idx], gb.at[b], gsems[b])

    def wait_gather(b):
      pltpu.make_async_copy(x_lo.at[pl.ds(0, _CHUNK)], gb.at[b],
                            gsems[b]).wait()

    def wait_scatter(b):
      pltpu.make_async_copy(gb.at[b], acc.at[pl.ds(0, _CHUNK)],
                            ssems[b]).wait()

    issue_cv(0, 0)
    issue_cv(1, 1)
    wait_cv(0)
    issue_gather(0)

    def chunk3(j, _):
      for b in range(_NBUF):
        jj = j * _NBUF + b
        nb = (b + 1) % _NBUF
        nnb = (b + 2) % _NBUF

        @pl.when(jj + 2 < n_chunks)
        def _():
          issue_cv(jj + 2, nnb)

        @pl.when(jnp.logical_and(jj + 1 < n_chunks, jj >= 2))
        def _():
          wait_scatter(nb)

        @pl.when(jj + 1 < n_chunks)
        def _():
          wait_cv(nb)
          issue_gather(nb)

        wait_gather(b)

        def scale_group(g, _):
          vals16 = lax.bitcast_convert_type(
              cv_v[b, 1, pl.ds(g * _LANES, _LANES)], jnp.float32)
          for e16 in range(_LANES):
            v = _bcast_lane(vals16, e16)
            row = g * _LANES + e16
            for q in range(128 // _LANES):
              sl = pl.ds(q * _LANES, _LANES)
              gb[b, row, sl] = gb[b, row, sl] * v
          return 0

        lax.fori_loop(0, _CHUNK // _LANES, scale_group, 0)
        pltpu.async_copy(gb.at[b], acc.at[rows_v.at[jj]], ssems[b], add=True)
      return 0

    lax.fori_loop(0, n_chunks // _NBUF, chunk3, 0)
    for b in range(_NBUF):
      wait_scatter(b)
    plsc.subcore_barrier()

    if not do_relu:
      @pl.when(cid == 0)
      def _():
        pltpu.sync_copy(acc.at[pl.ds(base_row, rows_per_tile)],
                        out_lo.at[pl.ds(base_row, rows_per_tile)])

      @pl.when(cid == 1)
      def _():
        pltpu.sync_copy(acc.at[pl.ds(base_row, rows_per_tile)],
                        out_hi.at[pl.ds(base_row, rows_per_tile)])
    else:
      for t in range(n_drain):
        row0 = base_row + t * drain
        pltpu.sync_copy(acc.at[pl.ds(row0, drain)], gb.at[0, pl.ds(0, drain)])

        def relu_row(i, _):
          for q in range(128 // _LANES):
            sl = pl.ds(q * _LANES, _LANES)
            gb[0, i, sl] = jnp.maximum(gb[0, i, sl], 0.0)
          return 0

        lax.fori_loop(0, drain, relu_row, 0)

        @pl.when(cid == 0)
        def _():
          pltpu.sync_copy(gb.at[0, pl.ds(0, drain)],
                          out_lo.at[pl.ds(row0, drain)])

        @pl.when(cid == 1)
        def _():
          pltpu.sync_copy(gb.at[0, pl.ds(0, drain)],
                          out_hi.at[pl.ds(row0, drain)])

  return spmm


def _dense_block(alo, ahi, blo, bhi, ulo, uhi, w_ref, olo, ohi):
  parts = (alo, ahi, blo, bhi, ulo, uhi)
  acc = jnp.zeros((alo.shape[0], 256), jnp.float32)
  for i, p in enumerate(parts):
    acc += jnp.dot(p[...], w_ref[pl.ds(i * 128, 128), :],
                   preferred_element_type=jnp.float32)
  acc = jnp.maximum(acc, 0.0)
  olo[...] = acc[:, :128]
  ohi[...] = acc[:, 128:]


def _dense(alo, ahi, blo, bhi, ulo, uhi, w):
  n = alo.shape[0]
  blk = 400
  grid = n // blk
  half = pl.BlockSpec((blk, 128), lambda i: (i, 0))
  return pl.pallas_call(
      _dense_block,
      grid=(grid,),
      in_specs=[half] * 6 + [pl.BlockSpec((768, 256), lambda i: (0, 0))],
      out_specs=[half, half],
      out_shape=(
          jax.ShapeDtypeStruct((n, 128), jnp.float32),
          jax.ShapeDtypeStruct((n, 128), jnp.float32),
      ),
  )(alo, ahi, blo, bhi, ulo, uhi, w)


def _prep_edges(rows, cols, vals, n_chunks):
  e = rows.shape[0]
  tot = _N_SUBCORES * n_chunks * _CHUNK
  pad = tot - e
  r = jnp.pad(rows.astype(jnp.int32), (0, pad)).reshape(
      _N_SUBCORES, n_chunks, _CHUNK)
  c = jnp.pad(cols.astype(jnp.int32), (0, pad)).reshape(
      _N_SUBCORES, n_chunks, 1, _CHUNK)
  v = lax.bitcast_convert_type(jnp.pad(vals, (0, pad)), jnp.int32).reshape(
      _N_SUBCORES, n_chunks, 1, _CHUNK)
  return r, jnp.concatenate([c, v], axis=2)


def kernel(user_emb, item_emb, weight0, weight1, social_edges, social_vals,
           inter_edges, inter_vals):
  n_u, d = user_emb.shape
  n_i = item_emb.shape[0]
  e = social_edges.shape[1]
  per_tile = -(-e // _N_SUBCORES)
  n_chunks = -(-per_tile // _CHUNK)
  n_chunks = -(-n_chunks // _NBUF) * _NBUF

  s_r, s_cv = _prep_edges(social_edges[0], social_edges[1], social_vals,
                          n_chunks)
  i_r, i_cv = _prep_edges(inter_edges[0], inter_edges[1], inter_vals,
                          n_chunks)
  t_r, t_cv = _prep_edges(inter_edges[1], inter_edges[0], inter_vals,
                          n_chunks)

  spmm = _make_spmm(n_u, n_chunks, do_relu=False)
  spmm_relu = _make_spmm(n_u, n_chunks, do_relu=True)
  zeros_h = jnp.zeros((640, 128), jnp.float32)

  u_lo, u_hi = user_emb[:, :128], user_emb[:, 128:]
  it_lo, it_hi = item_emb[:, :128], item_emb[:, 128:]

  outs_u = [(u_lo, u_hi)]
  outs_i = [(it_lo, it_hi)]
  for w in (weight0, weight1):
    a_lo, a_hi = spmm(u_lo, u_hi, zeros_h, s_r, s_cv)
    b_lo, b_hi = spmm(it_lo, it_hi, zeros_h, i_r, i_cv)
    it2_lo, it2_hi = spmm_relu(u_lo, u_hi, zeros_h, t_r, t_cv)
    a_lo, a_hi = a_lo[:n_u], a_hi[:n_u]
    b_lo, b_hi = b_lo[:n_u], b_hi[:n_u]
    new_it = (it2_lo[:n_i], it2_hi[:n_i])
    new_u = _dense(a_lo, a_hi, b_lo, b_hi, u_lo, u_hi, w)
    u_lo, u_hi = new_u
    it_lo, it_hi = new_it
    outs_u.append(new_u)
    outs_i.append(new_it)

  final_user = jnp.concatenate([h for pair in outs_u for h in pair], axis=1)
  final_item = jnp.concatenate([h for pair in outs_i for h in pair], axis=1)
  return final_user, final_item
